# baseline (speedup 1.0000x reference)
import functools
import jax
import jax.numpy as jnp
from jax import lax
from jax.experimental import pallas as pl
from jax.experimental.pallas import tpu as pltpu
from jax.experimental.pallas import tpu_sc as plsc

_B, _N, _C = 32, 8192, 128
_K = 64
_L = 16
_NW = 32
_CG = _C // _L
_TPW = _B * _CG // _NW
_RB = 2048
_NCH = _N // _RB
_CKB = 64
_CAP = 448
_TRIG = _CAP - _CKB
_UNR = 16


def _cmpex_stage(cand, j, k, n, alt=True):
    tj = 2 * j
    u = min(j, 16)
    nsub = j // u

    def body(m, _):
        g = m // nsub
        r0 = g * tj
        base = r0 + (m % nsub) * u
        wd = (r0 & (k & 63)) == 0
        if alt:
            wd = wd != ((r0 & 64) != 0)
        for lo in range(u):
            a = cand[base + lo]
            b = cand[base + lo + j]
            mx = jnp.maximum(a, b)
            mn = jnp.minimum(a, b)
            cand[base + lo] = jnp.where(wd, mx, mn)
            cand[base + lo + j] = jnp.where(wd, mn, mx)
        return 0

    lax.fori_loop(0, (n // tj) * nsub, body, 0)


def _sort128(cand):
    for k in (2, 4, 8, 16, 32, 64):
        j = k // 2
        while j >= 1:
            _cmpex_stage(cand, j, k, 128)
            j //= 2

    def fold(m, _):
        for lo in range(16):
            i = m * 16 + lo
            cand[i] = jnp.maximum(cand[i], cand[i + 64])
        return 0

    lax.fori_loop(0, 4, fold, 0)
    for j in (32, 16, 8, 4, 2, 1):
        _cmpex_stage(cand, j, 64, 64)


def _refil(cand, lo, hi, thr, cnt, lane, ninf):
    def body(i, c2):
        v = cand[i]
        cand[i] = ninf
        m = v > thr
        plsc.store_scatter(cand, [c2, lane], v, mask=m)
        return c2 + jnp.where(m, 1, 0)

    return lax.fori_loop(lo, hi, body, cnt)


def _sc_body(x_hbm, o_hbm, buf, cand):
    cid = lax.axis_index("c")
    sid = lax.axis_index("s")
    wid = sid * 2 + cid
    lane = lax.iota(jnp.int32, _L)
    ninf = jnp.full((_L,), -jnp.inf, jnp.float32)
    zero = jnp.zeros((_L,), jnp.int32)

    def compact(tc):
        thr, cnt = tc
        _sort128(cand)
        thr2 = jnp.maximum(thr, cand[63])
        cnt2 = _refil(cand, 0, 64, thr2, zero, lane, ninf)

        def clear(i, _):
            cand[64 + i] = ninf
            return 0

        lax.fori_loop(0, 64, clear, 0)
        cnt2 = _refil(cand, 128, _CAP, thr2, cnt2, lane, ninf)
        return thr2, cnt2

    def task_body(t, _):
        task = wid * _TPW + t
        b = task // _CG
        c0 = (task % _CG) * _L

        def init_row(i, _):
            cand[i] = ninf
            return 0

        lax.fori_loop(0, _CAP, init_row, 0)

        def block_body(blk, carry):
            def row_group(g, c2):
                thr2, cnt2 = c2
                rbase = blk * _CKB + g * _UNR
                for u in range(_UNR):
                    v = buf[rbase + u]
                    m = v > thr2
                    plsc.store_scatter(cand, [cnt2, lane], v, mask=m)
                    cnt2 = cnt2 + jnp.where(m, 1, 0)
                return thr2, cnt2

            carry = lax.fori_loop(0, _CKB // _UNR, row_group, carry)
            thr, cnt = carry
            return lax.cond(jnp.max(cnt) >= _TRIG, compact,
                            lambda tc: tc, (thr, cnt))

        def chunk_body(ch, carry):
            pltpu.sync_copy(
                x_hbm.at[b, pl.ds(ch * _RB, _RB), pl.ds(c0, _L)], buf)
            return lax.fori_loop(0, _RB // _CKB, block_body, carry)

        thr, cnt = lax.fori_loop(0, _NCH, chunk_body, (ninf, zero))

        thr, cnt = compact((thr, cnt))
        thr, cnt = lax.while_loop(
            lambda tc: jnp.max(tc[1]) > 127, compact, (thr, cnt))
        _sort128(cand)

        def fill_row(i, _):
            cand[64 + i] = thr
            return 0

        lax.fori_loop(0, _K, fill_row, 0)
        for j in (64, 32, 16, 8, 4, 2, 1):
            _cmpex_stage(cand, j, 64, 128, alt=False)

        pltpu.sync_copy(cand.at[pl.ds(0, _K)],
                        o_hbm.at[b, slice(None), pl.ds(c0, _L)])
        return 0

    lax.fori_loop(0, _TPW, task_body, 0)


@functools.cache
def _sc_topk():
    return pl.kernel(
        _sc_body,
        out_type=jax.ShapeDtypeStruct((_B, _K, _C), jnp.float32),
        mesh=plsc.VectorSubcoreMesh(
            core_axis_name="c", subcore_axis_name="s",
            num_cores=2, num_subcores=16),
        compiler_params=pltpu.CompilerParams(
            use_tc_tiling_on_sc=False, needs_layout_passes=False),
        scratch_types=[
            pltpu.VMEM((_RB, _L), jnp.float32),
            pltpu.VMEM((_CAP, _L), jnp.float32),
        ],
    )


def kernel(inputs):
    return _sc_topk()(inputs)


# speedup vs baseline: 2.4350x; 2.4350x over previous
import functools
import jax
import jax.numpy as jnp
from jax import lax
from jax.experimental import pallas as pl
from jax.experimental.pallas import tpu as pltpu
from jax.experimental.pallas import tpu_sc as plsc

_B, _N, _C = 32, 8192, 128
_K = 64
_L = 16
_NW = 32
_CG = _C // _L
_TPW = _B * _CG // _NW
_RB = 2048
_NCH = _N // _RB
_CKB = 64
_CAP = 192
_TRIG = 128
_UNR = 16


def _cmpex_stage(cand, j, k, n, alt=True):
    tj = 2 * j
    u = min(j, 16)
    nsub = j // u

    @plsc.parallel_loop(0, (n // tj) * nsub)
    def body(m):
        g = m // nsub
        r0 = g * tj
        base = r0 + (m % nsub) * u
        wd = (r0 & (k & 63)) == 0
        if alt:
            wd = wd != ((r0 & 64) != 0)
        for lo in range(u):
            ra = (base + lo) * _L
            rb = (base + lo + j) * _L
            a = cand[pl.ds(ra, _L)]
            b = cand[pl.ds(rb, _L)]
            mx = jnp.maximum(a, b)
            mn = jnp.minimum(a, b)
            cand[pl.ds(ra, _L)] = jnp.where(wd, mx, mn)
            cand[pl.ds(rb, _L)] = jnp.where(wd, mn, mx)


def _sort128(cand):
    for k in (2, 4, 8, 16, 32, 64):
        j = k // 2
        while j >= 1:
            _cmpex_stage(cand, j, k, 128)
            j //= 2

    @plsc.parallel_loop(0, 64, unroll=8)
    def fold(i):
        cand[pl.ds(i * _L, _L)] = jnp.maximum(
            cand[pl.ds(i * _L, _L)], cand[pl.ds((i + 64) * _L, _L)])
    for j in (32, 16, 8, 4, 2, 1):
        _cmpex_stage(cand, j, 64, 64)


def _refil(cand, lo, hi, thr, cnt, lane, ninf):
    @plsc.parallel_loop(lo, hi, unroll=8, carry=cnt)
    def body(i, c2):
        v = cand[pl.ds(i * _L, _L)]
        cand[pl.ds(i * _L, _L)] = ninf
        m = v > thr
        plsc.store_scatter(cand, [c2], v, mask=m)
        return c2 + jnp.where(m, _L, 0)

    return body


def _sc_body(x_hbm, o_hbm, buf, cand, out2, sem):
    cid = lax.axis_index("c")
    sid = lax.axis_index("s")
    wid = sid * 2 + cid
    lane = lax.iota(jnp.int32, _L)
    ninf = jnp.full((_L,), -jnp.inf, jnp.float32)
    zero = jnp.zeros((_L,), jnp.int32)

    def compact(tc):
        thr, cnt = tc
        _sort128(cand)
        thr2 = jnp.maximum(thr, cand[pl.ds(63 * _L, _L)])
        cnt2 = _refil(cand, 0, 64, thr2, lane, lane, ninf)

        @plsc.parallel_loop(0, 64, unroll=8)
        def clear(i):
            cand[pl.ds((64 + i) * _L, _L)] = ninf
        cnt2 = _refil(cand, 128, _CAP, thr2, cnt2, lane, ninf)
        return thr2, cnt2

    def task_body(t, _):
        task = wid * _TPW + t
        b = task // _CG
        c0 = (task % _CG) * _L

        @plsc.parallel_loop(0, _CAP, unroll=8)
        def init_row(i):
            cand[pl.ds(i * _L, _L)] = ninf

        def block_body(par, blk, carry):
            @plsc.parallel_loop(blk * _CKB, (blk + 1) * _CKB,
                                unroll=_UNR, carry=carry)
            def stream_row(i, c2):
                thr2, cnt2 = c2
                v = buf[par, i]
                m = v > thr2
                plsc.store_scatter(cand, [cnt2], v, mask=m)
                return thr2, cnt2 + jnp.where(m, _L, 0)

            thr, cnt = stream_row
            return lax.while_loop(
                lambda tc: jnp.max(tc[1]) >= _TRIG * _L, compact, (thr, cnt))

        def chunk_src(ch):
            return x_hbm.at[b, pl.ds(ch * _RB, _RB), pl.ds(c0, _L)]

        pltpu.async_copy(chunk_src(0), buf.at[0], sem.at[0])

        def chunk_body(ch, carry):
            par = lax.rem(ch, 2)

            @pl.when(ch + 1 < _NCH)
            def _():
                pltpu.async_copy(chunk_src(ch + 1), buf.at[1 - par],
                                 sem.at[1 - par])

            pltpu.make_async_copy(chunk_src(ch), buf.at[par],
                                  sem.at[par]).wait()
            return lax.fori_loop(0, _RB // _CKB,
                                 functools.partial(block_body, par), carry)

        thr, cnt = lax.fori_loop(0, _NCH, chunk_body, (ninf, lane))

        thr, cnt = compact((thr, cnt))
        _sort128(cand)

        @plsc.parallel_loop(0, _K, unroll=8)
        def fill_row(i):
            cand[pl.ds((64 + i) * _L, _L)] = thr
        for j in (64, 32, 16, 8, 4, 2, 1):
            _cmpex_stage(cand, j, 64, 128, alt=False)

        @plsc.parallel_loop(0, _K, unroll=8)
        def stage_out(i):
            out2[i] = cand[pl.ds(i * _L, _L)]

        pltpu.sync_copy(out2, o_hbm.at[b, slice(None), pl.ds(c0, _L)])
        return 0

    lax.fori_loop(0, _TPW, task_body, 0)


@functools.cache
def _sc_topk():
    return pl.kernel(
        _sc_body,
        out_type=jax.ShapeDtypeStruct((_B, _K, _C), jnp.float32),
        mesh=plsc.VectorSubcoreMesh(
            core_axis_name="c", subcore_axis_name="s",
            num_cores=2, num_subcores=16),
        compiler_params=pltpu.CompilerParams(
            use_tc_tiling_on_sc=False, needs_layout_passes=False),
        scratch_types=[
            pltpu.VMEM((2, _RB, _L), jnp.float32),
            pltpu.VMEM((_CAP * _L,), jnp.float32),
            pltpu.VMEM((_K, _L), jnp.float32),
            pltpu.SemaphoreType.DMA((2,)),
        ],
    )


def kernel(inputs):
    return _sc_topk()(inputs)


# speedup vs baseline: 2.7353x; 1.1233x over previous
import functools
import jax
import jax.numpy as jnp
from jax import lax
from jax.experimental import pallas as pl
from jax.experimental.pallas import tpu as pltpu
from jax.experimental.pallas import tpu_sc as plsc

_B, _N, _C = 32, 8192, 128
_K = 64
_L = 16
_NW = 32
_CG = _C // _L
_BSC = 28
_TPW = _BSC * _CG // _NW
_RB = 2048
_NCH = _N // _RB
_CKB = 64
_CAP = 192
_TRIG = 128
_UNR = 16


def _cmpex_stage(cand, j, k, n, alt=True):
    tj = 2 * j
    u = min(j, 16)
    nsub = j // u

    @plsc.parallel_loop(0, (n // tj) * nsub)
    def body(m):
        g = m // nsub
        r0 = g * tj
        base = r0 + (m % nsub) * u
        wd = (r0 & (k & 63)) == 0
        if alt:
            wd = wd != ((r0 & 64) != 0)
        for lo in range(u):
            ra = (base + lo) * _L
            rb = (base + lo + j) * _L
            a = cand[pl.ds(ra, _L)]
            b = cand[pl.ds(rb, _L)]
            mx = jnp.maximum(a, b)
            mn = jnp.minimum(a, b)
            cand[pl.ds(ra, _L)] = jnp.where(wd, mx, mn)
            cand[pl.ds(rb, _L)] = jnp.where(wd, mn, mx)


def _sort128(cand):
    for k in (2, 4, 8, 16, 32, 64):
        j = k // 2
        while j >= 1:
            _cmpex_stage(cand, j, k, 128)
            j //= 2

    @plsc.parallel_loop(0, 64, unroll=8)
    def fold(i):
        cand[pl.ds(i * _L, _L)] = jnp.maximum(
            cand[pl.ds(i * _L, _L)], cand[pl.ds((i + 64) * _L, _L)])
    for j in (32, 16, 8, 4, 2, 1):
        _cmpex_stage(cand, j, 64, 64)


def _refil(cand, lo, hi, thr, cnt, lane, ninf):
    @plsc.parallel_loop(lo, hi, unroll=8, carry=cnt)
    def body(i, c2):
        v = cand[pl.ds(i * _L, _L)]
        cand[pl.ds(i * _L, _L)] = ninf
        m = v > thr
        plsc.store_scatter(cand, [c2], v, mask=m)
        return c2 + jnp.where(m, _L, 0)

    return body


def _sc_body(x_hbm, o_hbm, buf, cand, out2, sem):
    cid = lax.axis_index("c")
    sid = lax.axis_index("s")
    wid = sid * 2 + cid
    lane = lax.iota(jnp.int32, _L)
    ninf = jnp.full((_L,), -jnp.inf, jnp.float32)
    zero = jnp.zeros((_L,), jnp.int32)

    def compact(tc):
        thr, cnt = tc
        _sort128(cand)
        thr2 = jnp.maximum(thr, cand[pl.ds(63 * _L, _L)])
        cnt2 = _refil(cand, 0, 64, thr2, lane, lane, ninf)

        @plsc.parallel_loop(0, 64, unroll=8)
        def clear(i):
            cand[pl.ds((64 + i) * _L, _L)] = ninf
        cnt2 = _refil(cand, 128, _CAP, thr2, cnt2, lane, ninf)
        return thr2, cnt2

    def task_body(t, _):
        task = wid * _TPW + t
        b = task // _CG
        c0 = (task % _CG) * _L

        @plsc.parallel_loop(0, _CAP, unroll=8)
        def init_row(i):
            cand[pl.ds(i * _L, _L)] = ninf

        def block_body(par, blk, carry):
            @plsc.parallel_loop(blk * _CKB, (blk + 1) * _CKB,
                                unroll=_UNR, carry=carry)
            def stream_row(i, c2):
                thr2, cnt2 = c2
                v = buf[par, i]
                m = v > thr2
                plsc.store_scatter(cand, [cnt2], v, mask=m)
                return thr2, cnt2 + jnp.where(m, _L, 0)

            thr, cnt = stream_row
            return lax.while_loop(
                lambda tc: jnp.max(tc[1]) >= _TRIG * _L, compact, (thr, cnt))

        def chunk_src(ch):
            return x_hbm.at[b, pl.ds(ch * _RB, _RB), pl.ds(c0, _L)]

        pltpu.async_copy(chunk_src(0), buf.at[0], sem.at[0])

        def chunk_body(ch, carry):
            par = lax.rem(ch, 2)

            @pl.when(ch + 1 < _NCH)
            def _():
                pltpu.async_copy(chunk_src(ch + 1), buf.at[1 - par],
                                 sem.at[1 - par])

            pltpu.make_async_copy(chunk_src(ch), buf.at[par],
                                  sem.at[par]).wait()
            return lax.fori_loop(0, _RB // _CKB,
                                 functools.partial(block_body, par), carry)

        thr, cnt = lax.fori_loop(0, _NCH, chunk_body, (ninf, lane))

        thr, cnt = compact((thr, cnt))
        _sort128(cand)

        @plsc.parallel_loop(0, _K, unroll=8)
        def fill_row(i):
            cand[pl.ds((64 + i) * _L, _L)] = thr
        for j in (64, 32, 16, 8, 4, 2, 1):
            _cmpex_stage(cand, j, 64, 128, alt=False)

        @plsc.parallel_loop(0, _K, unroll=8)
        def stage_out(i):
            out2[i] = cand[pl.ds(i * _L, _L)]

        pltpu.sync_copy(out2, o_hbm.at[b, slice(None), pl.ds(c0, _L)])
        return 0

    lax.fori_loop(0, _TPW, task_body, 0)


@functools.cache
def _sc_topk():
    return pl.kernel(
        _sc_body,
        out_type=jax.ShapeDtypeStruct((_BSC, _K, _C), jnp.float32),
        mesh=plsc.VectorSubcoreMesh(
            core_axis_name="c", subcore_axis_name="s",
            num_cores=2, num_subcores=16),
        compiler_params=pltpu.CompilerParams(
            use_tc_tiling_on_sc=False, needs_layout_passes=False),
        scratch_types=[
            pltpu.VMEM((2, _RB, _L), jnp.float32),
            pltpu.VMEM((_CAP * _L,), jnp.float32),
            pltpu.VMEM((_K, _L), jnp.float32),
            pltpu.SemaphoreType.DMA((2,)),
        ],
    )


def _tc_stage(x, j, want_desc_fn):
    s, lanes = x.shape
    if j >= 8:
        g = s // (2 * j)
        xr = x.reshape(g, 2, j, lanes)
        top = xr[:, 0]
        bot = xr[:, 1]
        r0 = jax.lax.broadcasted_iota(jnp.int32, (g, 1, 1), 0) * (2 * j)
        wd = want_desc_fn(r0)
        mx = jnp.maximum(top, bot)
        mn = jnp.minimum(top, bot)
        return jnp.stack([jnp.where(wd, mx, mn), jnp.where(wd, mn, mx)],
                         axis=1).reshape(s, lanes)
    else:
        r = jax.lax.broadcasted_iota(jnp.int32, (s, 1), 0)
        bitj = (r & j) != 0
        partner = jnp.where(bitj, jnp.roll(x, j, axis=0),
                            jnp.roll(x, -j, axis=0))
        take_max = want_desc_fn(r) ^ bitj
        return jnp.where(take_max, jnp.maximum(x, partner),
                         jnp.minimum(x, partner))


def _tc_body(x_ref, o_ref):
    x = x_ref[0]
    for k in (2, 4, 8, 16, 32, 64):
        wd = lambda r, k=k: ((r & (k & 63)) == 0) ^ ((r & 64) != 0)
        j = k // 2
        while j >= 1:
            x = _tc_stage(x, j, wd)
            j //= 2
    merge_wd = lambda r: (r & 64) == 0
    for _ in range(7):
        s, lanes = x.shape
        xr = x.reshape(s // 128, 2, 64, lanes)
        x = jnp.maximum(xr[:, 0], xr[:, 1]).reshape(s // 2, lanes)
        for j in (32, 16, 8, 4, 2, 1):
            x = _tc_stage(x, j, merge_wd)
    o_ref[0] = x


def _tc_topk(inputs):
    return pl.pallas_call(
        _tc_body,
        grid=(_B - _BSC,),
        in_specs=[pl.BlockSpec((1, _N, _C), lambda i: (i + _BSC, 0, 0))],
        out_specs=pl.BlockSpec((1, _K, _C), lambda i: (i, 0, 0)),
        out_shape=jax.ShapeDtypeStruct((_B - _BSC, _K, _C), inputs.dtype),
    )(inputs)


def kernel(inputs):
    sc_out = _sc_topk()(inputs)
    tc_out = _tc_topk(inputs)
    return jnp.concatenate([sc_out, tc_out], axis=0)


# speedup vs baseline: 2.7454x; 1.0037x over previous
import functools
import jax
import jax.numpy as jnp
from jax import lax
from jax.experimental import pallas as pl
from jax.experimental.pallas import tpu as pltpu
from jax.experimental.pallas import tpu_sc as plsc

_B, _N, _C = 32, 8192, 128
_K = 64
_L = 16
_NW = 32
_CG = _C // _L
_BSC = 26
_NT = _BSC * _CG
_TPW = -(-_NT // _NW)
_RB = 2048
_NCH = _N // _RB
_CKB = 64
_CAP = 192
_TRIG = 128
_UNR = 16


def _cmpex_stage(cand, j, k, n, alt=True):
    tj = 2 * j
    u = min(j, 16)
    nsub = j // u

    @plsc.parallel_loop(0, (n // tj) * nsub)
    def body(m):
        g = m // nsub
        r0 = g * tj
        base = r0 + (m % nsub) * u
        wd = (r0 & (k & 63)) == 0
        if alt:
            wd = wd != ((r0 & 64) != 0)
        for lo in range(u):
            ra = (base + lo) * _L
            rb = (base + lo + j) * _L
            a = cand[pl.ds(ra, _L)]
            b = cand[pl.ds(rb, _L)]
            mx = jnp.maximum(a, b)
            mn = jnp.minimum(a, b)
            cand[pl.ds(ra, _L)] = jnp.where(wd, mx, mn)
            cand[pl.ds(rb, _L)] = jnp.where(wd, mn, mx)


def _sort128(cand):
    for k in (2, 4, 8, 16, 32, 64):
        j = k // 2
        while j >= 1:
            _cmpex_stage(cand, j, k, 128)
            j //= 2

    @plsc.parallel_loop(0, 64, unroll=8)
    def fold(i):
        cand[pl.ds(i * _L, _L)] = jnp.maximum(
            cand[pl.ds(i * _L, _L)], cand[pl.ds((i + 64) * _L, _L)])
    for j in (32, 16, 8, 4, 2, 1):
        _cmpex_stage(cand, j, 64, 64)


def _refil(cand, lo, hi, thr, cnt, lane, ninf):
    @plsc.parallel_loop(lo, hi, unroll=8, carry=cnt)
    def body(i, c2):
        v = cand[pl.ds(i * _L, _L)]
        cand[pl.ds(i * _L, _L)] = ninf
        m = v > thr
        plsc.store_scatter(cand, [c2], v, mask=m)
        return c2 + jnp.where(m, _L, 0)

    return body


def _sc_body(x_hbm, o_hbm, buf, cand, out2, sem):
    cid = lax.axis_index("c")
    sid = lax.axis_index("s")
    wid = sid * 2 + cid
    lane = lax.iota(jnp.int32, _L)
    ninf = jnp.full((_L,), -jnp.inf, jnp.float32)
    zero = jnp.zeros((_L,), jnp.int32)

    def compact(tc):
        thr, cnt = tc
        _sort128(cand)
        thr2 = jnp.maximum(thr, cand[pl.ds(63 * _L, _L)])
        cnt2 = _refil(cand, 0, 64, thr2, lane, lane, ninf)

        @plsc.parallel_loop(0, 64, unroll=8)
        def clear(i):
            cand[pl.ds((64 + i) * _L, _L)] = ninf
        cnt2 = _refil(cand, 128, _CAP, thr2, cnt2, lane, ninf)
        return thr2, cnt2

    def task_body(t, _):
        task = t * _NW + wid

        @pl.when(task < _NT)
        def _():
            _one_task(task)

        return 0

    def _one_task(task):
        b = task // _CG
        c0 = (task % _CG) * _L

        @plsc.parallel_loop(0, _CAP, unroll=8)
        def init_row(i):
            cand[pl.ds(i * _L, _L)] = ninf

        def block_body(par, blk, carry):
            @plsc.parallel_loop(blk * _CKB, (blk + 1) * _CKB,
                                unroll=_UNR, carry=carry)
            def stream_row(i, c2):
                thr2, cnt2 = c2
                v = buf[par, i]
                m = v > thr2
                plsc.store_scatter(cand, [cnt2], v, mask=m)
                return thr2, cnt2 + jnp.where(m, _L, 0)

            thr, cnt = stream_row
            return lax.while_loop(
                lambda tc: jnp.max(tc[1]) >= _TRIG * _L, compact, (thr, cnt))

        def chunk_src(ch):
            return x_hbm.at[b, pl.ds(ch * _RB, _RB), pl.ds(c0, _L)]

        pltpu.async_copy(chunk_src(0), buf.at[0], sem.at[0])

        def chunk_body(ch, carry):
            par = lax.rem(ch, 2)

            @pl.when(ch + 1 < _NCH)
            def _():
                pltpu.async_copy(chunk_src(ch + 1), buf.at[1 - par],
                                 sem.at[1 - par])

            pltpu.make_async_copy(chunk_src(ch), buf.at[par],
                                  sem.at[par]).wait()
            return lax.fori_loop(0, _RB // _CKB,
                                 functools.partial(block_body, par), carry)

        thr, cnt = lax.fori_loop(0, _NCH, chunk_body, (ninf, lane))

        thr, cnt = compact((thr, cnt))
        _sort128(cand)

        @plsc.parallel_loop(0, _K, unroll=8)
        def fill_row(i):
            cand[pl.ds((64 + i) * _L, _L)] = thr
        for j in (64, 32, 16, 8, 4, 2, 1):
            _cmpex_stage(cand, j, 64, 128, alt=False)

        @plsc.parallel_loop(0, _K, unroll=8)
        def stage_out(i):
            out2[i] = cand[pl.ds(i * _L, _L)]

        pltpu.sync_copy(out2, o_hbm.at[b, slice(None), pl.ds(c0, _L)])

    lax.fori_loop(0, _TPW, task_body, 0)


@functools.cache
def _sc_topk():
    return pl.kernel(
        _sc_body,
        out_type=jax.ShapeDtypeStruct((_BSC, _K, _C), jnp.float32),
        mesh=plsc.VectorSubcoreMesh(
            core_axis_name="c", subcore_axis_name="s",
            num_cores=2, num_subcores=16),
        compiler_params=pltpu.CompilerParams(
            use_tc_tiling_on_sc=False, needs_layout_passes=False),
        scratch_types=[
            pltpu.VMEM((2, _RB, _L), jnp.float32),
            pltpu.VMEM((_CAP * _L,), jnp.float32),
            pltpu.VMEM((_K, _L), jnp.float32),
            pltpu.SemaphoreType.DMA((2,)),
        ],
    )


def _tc_stage(x, j, want_desc_fn):
    s, lanes = x.shape
    if j >= 8:
        g = s // (2 * j)
        xr = x.reshape(g, 2, j, lanes)
        top = xr[:, 0]
        bot = xr[:, 1]
        r0 = jax.lax.broadcasted_iota(jnp.int32, (g, 1, 1), 0) * (2 * j)
        wd = want_desc_fn(r0)
        mx = jnp.maximum(top, bot)
        mn = jnp.minimum(top, bot)
        return jnp.stack([jnp.where(wd, mx, mn), jnp.where(wd, mn, mx)],
                         axis=1).reshape(s, lanes)
    else:
        r = jax.lax.broadcasted_iota(jnp.int32, (s, 1), 0)
        bitj = (r & j) != 0
        partner = jnp.where(bitj, jnp.roll(x, j, axis=0),
                            jnp.roll(x, -j, axis=0))
        take_max = want_desc_fn(r) ^ bitj
        return jnp.where(take_max, jnp.maximum(x, partner),
                         jnp.minimum(x, partner))


def _tc_body(x_ref, o_ref):
    x = x_ref[0]
    for k in (2, 4, 8, 16, 32, 64):
        wd = lambda r, k=k: ((r & (k & 63)) == 0) ^ ((r & 64) != 0)
        j = k // 2
        while j >= 1:
            x = _tc_stage(x, j, wd)
            j //= 2
    merge_wd = lambda r: (r & 64) == 0
    for _ in range(7):
        s, lanes = x.shape
        xr = x.reshape(s // 128, 2, 64, lanes)
        x = jnp.maximum(xr[:, 0], xr[:, 1]).reshape(s // 2, lanes)
        for j in (32, 16, 8, 4, 2, 1):
            x = _tc_stage(x, j, merge_wd)
    o_ref[0] = x


def _tc_topk(inputs):
    return pl.pallas_call(
        _tc_body,
        grid=(_B - _BSC,),
        in_specs=[pl.BlockSpec((1, _N, _C), lambda i: (i + _BSC, 0, 0))],
        out_specs=pl.BlockSpec((1, _K, _C), lambda i: (i, 0, 0)),
        out_shape=jax.ShapeDtypeStruct((_B - _BSC, _K, _C), inputs.dtype),
    )(inputs)


def kernel(inputs):
    sc_out = _sc_topk()(inputs)
    tc_out = _tc_topk(inputs)
    return jnp.concatenate([sc_out, tc_out], axis=0)


# speedup vs baseline: 2.7473x; 1.0007x over previous
import functools
import jax
import jax.numpy as jnp
from jax import lax
from jax.experimental import pallas as pl
from jax.experimental.pallas import tpu as pltpu
from jax.experimental.pallas import tpu_sc as plsc

_B, _N, _C = 32, 8192, 128
_K = 64
_L = 16
_NW = 32
_CG = _C // _L
_BSC = 26
_NT = _BSC * _CG
_TPW = -(-_NT // _NW)
_RB = 2048
_NCH = _N // _RB
_CKB = 64
_CAP = 192
_TRIG = 128
_UNR = 16


def _cmpex_stage(cand, j, k, n, alt=True):
    tj = 2 * j
    u = min(j, 16)
    nsub = j // u

    @plsc.parallel_loop(0, (n // tj) * nsub)
    def body(m):
        g = m // nsub
        r0 = g * tj
        base = r0 + (m % nsub) * u
        wd = (r0 & (k & 63)) == 0
        if alt:
            wd = wd != ((r0 & 64) != 0)
        for lo in range(u):
            ra = (base + lo) * _L
            rb = (base + lo + j) * _L
            a = cand[pl.ds(ra, _L)]
            b = cand[pl.ds(rb, _L)]
            mx = jnp.maximum(a, b)
            mn = jnp.minimum(a, b)
            cand[pl.ds(ra, _L)] = jnp.where(wd, mx, mn)
            cand[pl.ds(rb, _L)] = jnp.where(wd, mn, mx)


def _sort128(cand):
    for k in (2, 4, 8, 16, 32, 64):
        j = k // 2
        while j >= 1:
            _cmpex_stage(cand, j, k, 128)
            j //= 2

    @plsc.parallel_loop(0, 64, unroll=8)
    def fold(i):
        cand[pl.ds(i * _L, _L)] = jnp.maximum(
            cand[pl.ds(i * _L, _L)], cand[pl.ds((i + 64) * _L, _L)])
    for j in (32, 16, 8, 4, 2, 1):
        _cmpex_stage(cand, j, 64, 64)


def _refil(cand, lo, hi, thr, cnt, lane, ninf):
    @plsc.parallel_loop(lo, hi, unroll=8, carry=cnt)
    def body(i, c2):
        v = cand[pl.ds(i * _L, _L)]
        cand[pl.ds(i * _L, _L)] = ninf
        m = v > thr
        plsc.store_scatter(cand, [c2], v, mask=m)
        return c2 + jnp.where(m, _L, 0)

    return body


def _sc_body(x_hbm, o_hbm, buf, cand, out2, sem):
    cid = lax.axis_index("c")
    sid = lax.axis_index("s")
    wid = sid * 2 + cid
    lane = lax.iota(jnp.int32, _L)
    ninf = jnp.full((_L,), -jnp.inf, jnp.float32)

    def compact(tc):
        thr, cnt = tc
        _sort128(cand)
        thr2 = jnp.maximum(thr, cand[pl.ds(63 * _L, _L)])
        cnt2 = _refil(cand, 0, 64, thr2, lane, lane, ninf)

        @plsc.parallel_loop(0, 64, unroll=8)
        def clear(i):
            cand[pl.ds((64 + i) * _L, _L)] = ninf
        cnt2 = _refil(cand, 128, _CAP, thr2, cnt2, lane, ninf)
        return thr2, cnt2

    def task_body(t, _):
        task = t * _NW + wid

        @pl.when(task < _NT)
        def _():
            _one_task(task)

        return 0

    def _one_task(task):
        b = task // _CG
        c0 = (task % _CG) * _L

        @plsc.parallel_loop(0, _CAP, unroll=8)
        def init_row(i):
            cand[pl.ds(i * _L, _L)] = ninf

        def block_body(par, blk, carry):
            @plsc.parallel_loop(blk * _CKB, (blk + 1) * _CKB,
                                unroll=_UNR, carry=carry)
            def stream_row(i, c2):
                thr2, cnt2 = c2
                v = buf[par, i]
                m = v > thr2
                plsc.store_scatter(cand, [cnt2], v, mask=m)
                return thr2, cnt2 + jnp.where(m, _L, 0)

            thr, cnt = stream_row
            return lax.while_loop(
                lambda tc: jnp.max(tc[1]) >= _TRIG * _L, compact, (thr, cnt))

        def chunk_src(ch):
            return x_hbm.at[b, pl.ds(ch * _RB, _RB), pl.ds(c0, _L)]

        pltpu.async_copy(chunk_src(0), buf.at[0], sem.at[0])

        def chunk_body(ch, carry):
            par = lax.rem(ch, 2)

            @pl.when(ch + 1 < _NCH)
            def _():
                pltpu.async_copy(chunk_src(ch + 1), buf.at[1 - par],
                                 sem.at[1 - par])

            pltpu.make_async_copy(chunk_src(ch), buf.at[par],
                                  sem.at[par]).wait()
            return lax.fori_loop(0, _RB // _CKB,
                                 functools.partial(block_body, par), carry)

        thr, cnt = lax.fori_loop(0, _NCH, chunk_body, (ninf, lane))

        thr, cnt = compact((thr, cnt))
        _sort128(cand)

        @plsc.parallel_loop(0, _K, unroll=8)
        def fill_row(i):
            cand[pl.ds((64 + i) * _L, _L)] = thr
        for j in (64, 32, 16, 8, 4, 2, 1):
            _cmpex_stage(cand, j, 64, 128, alt=False)

        @plsc.parallel_loop(0, _K, unroll=8)
        def stage_out(i):
            out2[i] = cand[pl.ds(i * _L, _L)]

        pltpu.sync_copy(out2, o_hbm.at[b, slice(None), pl.ds(c0, _L)])

    lax.fori_loop(0, _TPW, task_body, 0)


@functools.cache
def _sc_topk():
    return pl.kernel(
        _sc_body,
        out_type=jax.ShapeDtypeStruct((_BSC, _K, _C), jnp.float32),
        mesh=plsc.VectorSubcoreMesh(
            core_axis_name="c", subcore_axis_name="s",
            num_cores=2, num_subcores=16),
        compiler_params=pltpu.CompilerParams(
            use_tc_tiling_on_sc=False, needs_layout_passes=False),
        scratch_types=[
            pltpu.VMEM((2, _RB, _L), jnp.float32),
            pltpu.VMEM((_CAP * _L,), jnp.float32),
            pltpu.VMEM((_K, _L), jnp.float32),
            pltpu.SemaphoreType.DMA((2,)),
        ],
    )


def _tc_stage(x, j, want_desc_fn):
    s, lanes = x.shape
    if j >= 8:
        g = s // (2 * j)
        xr = x.reshape(g, 2, j, lanes)
        top = xr[:, 0]
        bot = xr[:, 1]
        r0 = jax.lax.broadcasted_iota(jnp.int32, (g, 1, 1), 0) * (2 * j)
        wd = want_desc_fn(r0)
        mx = jnp.maximum(top, bot)
        mn = jnp.minimum(top, bot)
        return jnp.stack([jnp.where(wd, mx, mn), jnp.where(wd, mn, mx)],
                         axis=1).reshape(s, lanes)
    else:
        r = jax.lax.broadcasted_iota(jnp.int32, (s, 1), 0)
        bitj = (r & j) != 0
        partner = jnp.where(bitj, jnp.roll(x, j, axis=0),
                            jnp.roll(x, -j, axis=0))
        take_max = want_desc_fn(r) ^ bitj
        return jnp.where(take_max, jnp.maximum(x, partner),
                         jnp.minimum(x, partner))


def _tc_body(x_ref, o_ref):
    x = x_ref[0]
    for k in (2, 4, 8, 16, 32, 64):
        wd = lambda r, k=k: ((r & (k & 63)) == 0) ^ ((r & 64) != 0)
        j = k // 2
        while j >= 1:
            x = _tc_stage(x, j, wd)
            j //= 2
    merge_wd = lambda r: (r & 64) == 0
    for _ in range(7):
        s, lanes = x.shape
        xr = x.reshape(s // 128, 2, 64, lanes)
        x = jnp.maximum(xr[:, 0], xr[:, 1]).reshape(s // 2, lanes)
        for j in (32, 16, 8, 4, 2, 1):
            x = _tc_stage(x, j, merge_wd)
    o_ref[0] = x


def _tc_topk(inputs):
    return pl.pallas_call(
        _tc_body,
        grid=(_B - _BSC,),
        in_specs=[pl.BlockSpec((1, _N, _C), lambda i: (i + _BSC, 0, 0))],
        out_specs=pl.BlockSpec((1, _K, _C), lambda i: (i, 0, 0)),
        out_shape=jax.ShapeDtypeStruct((_B - _BSC, _K, _C), inputs.dtype),
    )(inputs)


def kernel(inputs):
    sc_out = _sc_topk()(inputs)
    tc_out = _tc_topk(inputs)
    return jnp.concatenate([sc_out, tc_out], axis=0)
